# baseline (speedup 1.0000x reference)
import functools

import functools

import jax
import jax.numpy as jnp
from jax import lax
from jax.experimental import pallas as pl
from jax.experimental.pallas import tpu as pltpu
from jax.experimental.pallas import tpu_sc as plsc

E = 16
DIM = 1024
HID = 512
N = 4096
K = 2
NK = N * K
T = 256
NT = NK // T
G = NT + E - 1


NW = 32
TPW = N // NW
CTOK = 32
NCK = TPW // CTOK
CTC = 16
NCC = TPW // CTC


@functools.cache
def _sc_mesh():
    return plsc.VectorSubcoreMesh(core_axis_name="c", subcore_axis_name="s")


def _dispatch_body(x_hbm, inv0_hbm, inv1_hbm, sc0_hbm, sc1_hbm,
                   rx_hbm, ss_hbm,
                   idx0_v, idx1_v, s0_v, s1_v, bufs, seml, sems):
    wid = lax.axis_index("s") * 2 + lax.axis_index("c")
    t0 = wid * TPW
    for c in range(NCK):
        pltpu.sync_copy(inv0_hbm.at[pl.ds(t0 + c * CTOK, CTOK)], idx0_v.at[c])
        pltpu.sync_copy(inv1_hbm.at[pl.ds(t0 + c * CTOK, CTOK)], idx1_v.at[c])
        pltpu.sync_copy(sc0_hbm.at[pl.ds(t0 + c * CTOK, CTOK)], s0_v.at[c])
        pltpu.sync_copy(sc1_hbm.at[pl.ds(t0 + c * CTOK, CTOK)], s1_v.at[c])
    pltpu.sync_copy(x_hbm.at[pl.ds(t0, CTOK), :], bufs[0])
    prev = []
    for c in range(NCK):
        cur = c & 1
        for cp in prev:
            cp.wait()
        ld = None
        if c + 1 < NCK:
            ld = pltpu.async_copy(
                x_hbm.at[pl.ds(t0 + (c + 1) * CTOK, CTOK), :],
                bufs[1 - cur], seml)
        prev = [
            pltpu.async_copy(bufs[cur], rx_hbm.at[idx0_v.at[c]], sems),
            pltpu.async_copy(bufs[cur], rx_hbm.at[idx1_v.at[c]], sems),
            pltpu.async_copy(s0_v.at[c], ss_hbm.at[idx0_v.at[c]], sems),
            pltpu.async_copy(s1_v.at[c], ss_hbm.at[idx1_v.at[c]], sems),
        ]
        if ld is not None:
            ld.wait()
    for cp in prev:
        cp.wait()


@functools.cache
def _build_dispatch():
    return pl.kernel(
        _dispatch_body,
        out_type=(jax.ShapeDtypeStruct((NK, DIM), jnp.float32),
                  jax.ShapeDtypeStruct((NK,), jnp.float32)),
        mesh=_sc_mesh(),
        scratch_types=[
            pltpu.VMEM((NCK, CTOK), jnp.int32),
            pltpu.VMEM((NCK, CTOK), jnp.int32),
            pltpu.VMEM((NCK, CTOK), jnp.float32),
            pltpu.VMEM((NCK, CTOK), jnp.float32),
            [pltpu.VMEM((CTOK, DIM), jnp.float32),
             pltpu.VMEM((CTOK, DIM), jnp.float32)],
            pltpu.SemaphoreType.DMA,
            pltpu.SemaphoreType.DMA,
        ],
    )


def _combine_body(ro_hbm, inv0_hbm, inv1_hbm, out_hbm,
                  idx0_v, idx1_v, bufa, bufb, outb, semg, semo):
    wid = lax.axis_index("s") * 2 + lax.axis_index("c")
    t0 = wid * TPW
    for c in range(NCC):
        pltpu.sync_copy(inv0_hbm.at[pl.ds(t0 + c * CTC, CTC)], idx0_v.at[c])
        pltpu.sync_copy(inv1_hbm.at[pl.ds(t0 + c * CTC, CTC)], idx1_v.at[c])
    gath = [pltpu.async_copy(ro_hbm.at[idx0_v.at[0]], bufa[0], semg),
            pltpu.async_copy(ro_hbm.at[idx1_v.at[0]], bufb[0], semg)]
    st = [None, None]
    for c in range(NCC):
        cur = c & 1
        nxt = 1 - cur
        ngath = None
        if c + 1 < NCC:
            ngath = [
                pltpu.async_copy(ro_hbm.at[idx0_v.at[c + 1]], bufa[nxt], semg),
                pltpu.async_copy(ro_hbm.at[idx1_v.at[c + 1]], bufb[nxt], semg),
            ]
        for cp in gath:
            cp.wait()
        if st[cur] is not None:
            st[cur].wait()
        ba, bb, ob = bufa[cur], bufb[cur], outb[cur]

        @plsc.parallel_loop(0, CTC * (DIM // 16), unroll=8)
        def add_slice(j):
            i = j // (DIM // 16)
            d = (j % (DIM // 16)) * 16
            ob[i, pl.ds(d, 16)] = ba[i, pl.ds(d, 16)] + bb[i, pl.ds(d, 16)]

        st[cur] = pltpu.async_copy(
            ob, out_hbm.at[pl.ds(t0 + c * CTC, CTC), :], semo)
        if ngath is not None:
            gath = ngath
    for s in st:
        if s is not None:
            s.wait()


@functools.cache
def _build_combine():
    return pl.kernel(
        _combine_body,
        out_type=jax.ShapeDtypeStruct((N, DIM), jnp.float32),
        mesh=_sc_mesh(),
        scratch_types=[
            pltpu.VMEM((NCC, CTC), jnp.int32),
            pltpu.VMEM((NCC, CTC), jnp.int32),
            [pltpu.VMEM((CTC, DIM), jnp.float32),
             pltpu.VMEM((CTC, DIM), jnp.float32)],
            [pltpu.VMEM((CTC, DIM), jnp.float32),
             pltpu.VMEM((CTC, DIM), jnp.float32)],
            [pltpu.VMEM((CTC, DIM), jnp.float32),
             pltpu.VMEM((CTC, DIM), jnp.float32)],
            pltpu.SemaphoreType.DMA,
            pltpu.SemaphoreType.DMA,
        ],
    )


def _swiglu_body(meta_ref, x_ref, sc_ref, w1_ref, w3_ref, w2_ref, out_ref):
    g = pl.program_id(0)
    xb = x_ref[...]
    w1e = w1_ref[0]
    w3e = w3_ref[0]
    w2e = w2_ref[0]
    a = jax.lax.dot_general(xb, w1e, (((1,), (1,)), ((), ())),
                            preferred_element_type=jnp.float32)
    b = jax.lax.dot_general(xb, w3e, (((1,), (1,)), ((), ())),
                            preferred_element_type=jnp.float32)
    h = (a * jax.nn.sigmoid(a)) * b
    o = jax.lax.dot_general(h, w2e, (((1,), (1,)), ((), ())),
                            preferred_element_type=jnp.float32)
    o = o * sc_ref[...]
    rows = jax.lax.broadcasted_iota(jnp.int32, (T, 1), 0)
    mask = (rows >= meta_ref[2, g]) & (rows < meta_ref[3, g])
    out_ref[...] = jnp.where(mask, o, out_ref[...])


def _grouped_swiglu(rx, ss, w1, w3, w2, meta):
    grid_spec = pltpu.PrefetchScalarGridSpec(
        num_scalar_prefetch=1,
        grid=(G,),
        in_specs=[
            pl.BlockSpec((T, DIM), lambda g, meta: (meta[0, g], 0)),
            pl.BlockSpec((T, 1), lambda g, meta: (meta[0, g], 0)),
            pl.BlockSpec((1, HID, DIM), lambda g, meta: (meta[1, g], 0, 0)),
            pl.BlockSpec((1, HID, DIM), lambda g, meta: (meta[1, g], 0, 0)),
            pl.BlockSpec((1, DIM, HID), lambda g, meta: (meta[1, g], 0, 0)),
        ],
        out_specs=pl.BlockSpec((T, DIM), lambda g, meta: (meta[0, g], 0)),
    )
    return pl.pallas_call(
        _swiglu_body,
        grid_spec=grid_spec,
        out_shape=jax.ShapeDtypeStruct((NK, DIM), jnp.float32),
    )(meta, rx, ss, w1, w3, w2)


CH = 128
NCH = N // CH


def _routing_body(sei_ref, ts_ref, inv0_ref, inv1_ref, sc0_ref, sc1_ref,
                  meta_ref):
    eids = jax.lax.broadcasted_iota(jnp.int32, (1, E), 1)

    def cnt_body(c, tot):
        blk = sei_ref[pl.ds(c * CH, CH), :]
        oh0 = (blk[:, 0:1] == eids).astype(jnp.int32)
        oh1 = (blk[:, 1:2] == eids).astype(jnp.int32)
        return tot + jnp.sum(oh0 + oh1, axis=0, keepdims=True)

    tot_row = jax.lax.fori_loop(0, NCH, cnt_body,
                                jnp.zeros((1, E), jnp.int32))
    tot_col = jnp.reshape(tot_row, (E, 1))

    er = jax.lax.broadcasted_iota(jnp.int32, (E, E), 0)
    ec = jax.lax.broadcasted_iota(jnp.int32, (E, E), 1)
    mask_lt = (er < ec).astype(jnp.float32)
    off_lo_row = jax.lax.dot_general(
        tot_col.astype(jnp.float32), mask_lt, (((0,), (0,)), ((), ())),
        preferred_element_type=jnp.float32,
        precision=jax.lax.Precision.HIGHEST)
    off_lo_col = jnp.reshape(off_lo_row, (E, 1)).astype(jnp.int32)
    off_hi_col = off_lo_col + tot_col

    r_i = jax.lax.broadcasted_iota(jnp.int32, (CH, CH), 0)
    c_i = jax.lax.broadcasted_iota(jnp.int32, (CH, CH), 1)
    tril_s = (c_i < r_i).astype(jnp.float32)
    base_row = off_lo_row

    def pb(c, carry):
        blk = sei_ref[pl.ds(c * CH, CH), :]
        oh0i = (blk[:, 0:1] == eids).astype(jnp.int32)
        oh1i = (blk[:, 1:2] == eids).astype(jnp.int32)
        oh0 = oh0i.astype(jnp.float32)
        oh1 = oh1i.astype(jnp.float32)
        A = jax.lax.dot_general(tril_s, oh0, (((1,), (0,)), ((), ())),
                                preferred_element_type=jnp.float32,
                                precision=jax.lax.Precision.HIGHEST)
        B = jax.lax.dot_general(tril_s, oh1, (((1,), (0,)), ((), ())),
                                preferred_element_type=jnp.float32,
                                precision=jax.lax.Precision.HIGHEST)
        base = base_row + carry.astype(jnp.float32)
        p0 = jnp.sum(oh0 * (base + A + B), axis=1, keepdims=True)
        p1 = jnp.sum(oh1 * (base + A + oh0 + B), axis=1, keepdims=True)
        inv0_ref[pl.ds(c * CH, CH), :] = p0.astype(jnp.int32)
        inv1_ref[pl.ds(c * CH, CH), :] = p1.astype(jnp.int32)
        return carry + jnp.sum(oh0i + oh1i, axis=0, keepdims=True)

    jax.lax.fori_loop(0, NCH, pb, jnp.zeros((1, E), jnp.int32))
    sc0_ref[...] = ts_ref[:, 0:1]
    sc1_ref[...] = ts_ref[:, 1:2]

    first_col = off_lo_col // T
    last_col = (off_hi_col - 1) // T
    tiles_col = jnp.where(tot_col > 0, last_col - first_col + 1, 0)
    mask_le_col = (ec <= er).astype(jnp.float32)
    cum_col = jax.lax.dot_general(
        mask_le_col, tiles_col.astype(jnp.float32),
        (((1,), (0,)), ((), ())),
        preferred_element_type=jnp.float32,
        precision=jax.lax.Precision.HIGHEST).astype(jnp.int32)
    total_b = cum_col[E - 1:E, :]

    grow = jax.lax.broadcasted_iota(jnp.int32, (1, 128), 1)
    ge_mask = (cum_col <= grow).astype(jnp.int32)
    e_of_g = jnp.sum(ge_mask, axis=0, keepdims=True)
    e_cl = jnp.minimum(e_of_g, E - 1)
    ecol = jax.lax.broadcasted_iota(jnp.int32, (E, 128), 0)
    ohg = (ecol == e_cl).astype(jnp.int32)

    def lk(v_col):
        return jnp.sum(ohg * v_col, axis=0, keepdims=True)

    first_g = lk(first_col)
    tiles_g = lk(tiles_col)
    cum_g = lk(cum_col)
    lo_g = lk(off_lo_col)
    hi_g = lk(off_hi_col)
    local = grow - (cum_g - tiles_g)
    t_g = first_g + local
    valid = grow < total_b
    tt = jnp.where(valid, t_g, NT - 1)
    eee = jnp.where(valid, e_cl, E - 1)
    st = jnp.where(valid, jnp.clip(lo_g - tt * T, 0, T), 0)
    en = jnp.where(valid, jnp.clip(hi_g - tt * T, 0, T), 0)
    meta_ref[0:1, :] = tt
    meta_ref[1:2, :] = eee
    meta_ref[2:3, :] = st
    meta_ref[3:4, :] = en
    meta_ref[4:5, :] = jnp.zeros((1, 128), jnp.int32)
    meta_ref[5:6, :] = jnp.zeros((1, 128), jnp.int32)
    meta_ref[6:7, :] = jnp.zeros((1, 128), jnp.int32)
    meta_ref[7:8, :] = jnp.zeros((1, 128), jnp.int32)


def _routing_tc(sei, ts):
    return pl.pallas_call(
        _routing_body,
        grid=(1,),
        in_specs=[
            pl.BlockSpec((N, K), lambda g: (0, 0)),
            pl.BlockSpec((N, K), lambda g: (0, 0)),
        ],
        out_specs=[
            pl.BlockSpec((N, 1), lambda g: (0, 0)),
            pl.BlockSpec((N, 1), lambda g: (0, 0)),
            pl.BlockSpec((N, 1), lambda g: (0, 0)),
            pl.BlockSpec((N, 1), lambda g: (0, 0)),
            pl.BlockSpec((8, 128), lambda g: (0, 0)),
        ],
        out_shape=[
            jax.ShapeDtypeStruct((N, 1), jnp.int32),
            jax.ShapeDtypeStruct((N, 1), jnp.int32),
            jax.ShapeDtypeStruct((N, 1), jnp.float32),
            jax.ShapeDtypeStruct((N, 1), jnp.float32),
            jax.ShapeDtypeStruct((8, 128), jnp.int32),
        ],
    )(sei, ts)


def _meta_jax(flat_exp):
    sizes = jnp.bincount(flat_exp, length=E).astype(jnp.int32)
    off = jnp.concatenate([jnp.zeros((1,), jnp.int32),
                           jnp.cumsum(sizes).astype(jnp.int32)])
    first_tile = off[:E] // T
    last_tile = (off[1:] - 1) // T
    tiles_e = jnp.where(sizes > 0, last_tile - first_tile + 1, 0).astype(jnp.int32)
    cum = jnp.cumsum(tiles_e)
    total = cum[-1]
    gids = jnp.arange(G, dtype=jnp.int32)
    e_of_g = jnp.searchsorted(cum, gids, side="right").astype(jnp.int32)
    valid = gids < total
    e_cl = jnp.minimum(e_of_g, E - 1)
    local = gids - (cum[e_cl] - tiles_e[e_cl])
    t_of_g = jnp.where(valid, first_tile[e_cl] + local, NT - 1).astype(jnp.int32)
    ee = jnp.where(valid, e_cl, E - 1).astype(jnp.int32)
    st = jnp.where(valid, jnp.clip(off[e_cl] - t_of_g * T, 0, T), 0).astype(jnp.int32)
    en = jnp.where(valid, jnp.clip(off[e_cl + 1] - t_of_g * T, 0, T), 0).astype(jnp.int32)
    meta = jnp.zeros((8, 128), jnp.int32)
    meta = meta.at[0, :G].set(t_of_g).at[1, :G].set(ee)
    meta = meta.at[2, :G].set(st).at[3, :G].set(en)
    return meta


def kernel(x, top_scores, selected_experts_indices, w1, w2, w3):
    inv0, inv1, sc0, sc1, meta = _routing_tc(selected_experts_indices,
                                             top_scores)
    inv0 = inv0.reshape(N)
    inv1 = inv1.reshape(N)

    rx, ss = _build_dispatch()(x, inv0, inv1, sc0.reshape(N), sc1.reshape(N))

    ro = _grouped_swiglu(rx, ss.reshape(NK, 1), w1, w3, w2, meta)

    out = _build_combine()(ro, inv0, inv1)
    return out


# speedup vs baseline: 1.1765x; 1.1765x over previous
import functools

import functools

import jax
import jax.numpy as jnp
from jax import lax
from jax.experimental import pallas as pl
from jax.experimental.pallas import tpu as pltpu
from jax.experimental.pallas import tpu_sc as plsc

E = 16
DIM = 1024
HID = 512
N = 4096
K = 2
NK = N * K
T = 256
NT = NK // T
G = NT + E - 1


NW = 32
TPW = N // NW
CTOK = 32
NCK = TPW // CTOK
CTC = 16
NCC = TPW // CTC


@functools.cache
def _sc_mesh():
    return plsc.VectorSubcoreMesh(core_axis_name="c", subcore_axis_name="s")


def _dispatch_body(x_hbm, inv0_hbm, inv1_hbm, rx_hbm,
                   idx0_v, idx1_v, bufs, seml, sems):
    wid = lax.axis_index("s") * 2 + lax.axis_index("c")
    t0 = wid * TPW
    for c in range(NCK):
        pltpu.sync_copy(inv0_hbm.at[pl.ds(t0 + c * CTOK, CTOK)], idx0_v.at[c])
        pltpu.sync_copy(inv1_hbm.at[pl.ds(t0 + c * CTOK, CTOK)], idx1_v.at[c])
    pltpu.sync_copy(x_hbm.at[pl.ds(t0, CTOK), :], bufs[0])
    prev = []
    for c in range(NCK):
        cur = c & 1
        for cp in prev:
            cp.wait()
        ld = None
        if c + 1 < NCK:
            ld = pltpu.async_copy(
                x_hbm.at[pl.ds(t0 + (c + 1) * CTOK, CTOK), :],
                bufs[1 - cur], seml)
        prev = [
            pltpu.async_copy(bufs[cur], rx_hbm.at[idx0_v.at[c]], sems),
            pltpu.async_copy(bufs[cur], rx_hbm.at[idx1_v.at[c]], sems),
        ]
        if ld is not None:
            ld.wait()
    for cp in prev:
        cp.wait()


@functools.cache
def _build_dispatch():
    return pl.kernel(
        _dispatch_body,
        out_type=jax.ShapeDtypeStruct((NK, DIM), jnp.float32),
        mesh=_sc_mesh(),
        scratch_types=[
            pltpu.VMEM((NCK, CTOK), jnp.int32),
            pltpu.VMEM((NCK, CTOK), jnp.int32),
            [pltpu.VMEM((CTOK, DIM), jnp.float32),
             pltpu.VMEM((CTOK, DIM), jnp.float32)],
            pltpu.SemaphoreType.DMA,
            pltpu.SemaphoreType.DMA,
        ],
    )


def _combine_body(ro_hbm, inv0_hbm, inv1_hbm, out_hbm,
                  idx0_v, idx1_v, bufa, bufb, outb, semg, semo):
    wid = lax.axis_index("s") * 2 + lax.axis_index("c")
    t0 = wid * TPW
    for c in range(NCC):
        pltpu.sync_copy(inv0_hbm.at[pl.ds(t0 + c * CTC, CTC)], idx0_v.at[c])
        pltpu.sync_copy(inv1_hbm.at[pl.ds(t0 + c * CTC, CTC)], idx1_v.at[c])
    gath = [pltpu.async_copy(ro_hbm.at[idx0_v.at[0]], bufa[0], semg),
            pltpu.async_copy(ro_hbm.at[idx1_v.at[0]], bufb[0], semg)]
    st = [None, None]
    for c in range(NCC):
        cur = c & 1
        nxt = 1 - cur
        ngath = None
        if c + 1 < NCC:
            ngath = [
                pltpu.async_copy(ro_hbm.at[idx0_v.at[c + 1]], bufa[nxt], semg),
                pltpu.async_copy(ro_hbm.at[idx1_v.at[c + 1]], bufb[nxt], semg),
            ]
        for cp in gath:
            cp.wait()
        if st[cur] is not None:
            st[cur].wait()
        ba, bb, ob = bufa[cur], bufb[cur], outb[cur]

        @plsc.parallel_loop(0, CTC * (DIM // 16), unroll=8)
        def add_slice(j):
            i = j // (DIM // 16)
            d = (j % (DIM // 16)) * 16
            ob[i, pl.ds(d, 16)] = ba[i, pl.ds(d, 16)] + bb[i, pl.ds(d, 16)]

        st[cur] = pltpu.async_copy(
            ob, out_hbm.at[pl.ds(t0 + c * CTC, CTC), :], semo)
        if ngath is not None:
            gath = ngath
    for s in st:
        if s is not None:
            s.wait()


@functools.cache
def _build_combine():
    return pl.kernel(
        _combine_body,
        out_type=jax.ShapeDtypeStruct((N, DIM), jnp.float32),
        mesh=_sc_mesh(),
        scratch_types=[
            pltpu.VMEM((NCC, CTC), jnp.int32),
            pltpu.VMEM((NCC, CTC), jnp.int32),
            [pltpu.VMEM((CTC, DIM), jnp.float32),
             pltpu.VMEM((CTC, DIM), jnp.float32)],
            [pltpu.VMEM((CTC, DIM), jnp.float32),
             pltpu.VMEM((CTC, DIM), jnp.float32)],
            [pltpu.VMEM((CTC, DIM), jnp.float32),
             pltpu.VMEM((CTC, DIM), jnp.float32)],
            pltpu.SemaphoreType.DMA,
            pltpu.SemaphoreType.DMA,
        ],
    )


def _swiglu_body(meta_ref, x_ref, sc_ref, w1_ref, w3_ref, w2_ref, out_ref):
    g = pl.program_id(0)
    xb = x_ref[...]
    w1e = w1_ref[0]
    w3e = w3_ref[0]
    w2e = w2_ref[0]
    a = jax.lax.dot_general(xb, w1e, (((1,), (1,)), ((), ())),
                            preferred_element_type=jnp.float32)
    b = jax.lax.dot_general(xb, w3e, (((1,), (1,)), ((), ())),
                            preferred_element_type=jnp.float32)
    h = (a * jax.nn.sigmoid(a)) * b
    o = jax.lax.dot_general(h, w2e, (((1,), (1,)), ((), ())),
                            preferred_element_type=jnp.float32)
    o = o * sc_ref[...]
    rows = jax.lax.broadcasted_iota(jnp.int32, (T, 1), 0)
    mask = (rows >= meta_ref[2, g]) & (rows < meta_ref[3, g])
    out_ref[...] = jnp.where(mask, o, out_ref[...])


def _grouped_swiglu(rx, ss, w1, w3, w2, meta):
    grid_spec = pltpu.PrefetchScalarGridSpec(
        num_scalar_prefetch=1,
        grid=(G,),
        in_specs=[
            pl.BlockSpec((T, DIM), lambda g, meta: (meta[0, g], 0)),
            pl.BlockSpec((T, 1), lambda g, meta: (meta[0, g], 0)),
            pl.BlockSpec((1, HID, DIM), lambda g, meta: (meta[1, g], 0, 0)),
            pl.BlockSpec((1, HID, DIM), lambda g, meta: (meta[1, g], 0, 0)),
            pl.BlockSpec((1, DIM, HID), lambda g, meta: (meta[1, g], 0, 0)),
        ],
        out_specs=pl.BlockSpec((T, DIM), lambda g, meta: (meta[0, g], 0)),
    )
    return pl.pallas_call(
        _swiglu_body,
        grid_spec=grid_spec,
        out_shape=jax.ShapeDtypeStruct((NK, DIM), jnp.float32),
    )(meta, rx, ss, w1, w3, w2)


CH = 128
NCH = N // CH


def _routing_body(sei_ref, ts_ref, inv0_ref, inv1_ref, sc0_ref, sc1_ref,
                  meta_ref):
    eids = jax.lax.broadcasted_iota(jnp.int32, (1, E), 1)

    def cnt_body(c, tot):
        blk = sei_ref[pl.ds(c * CH, CH), :]
        oh0 = (blk[:, 0:1] == eids).astype(jnp.int32)
        oh1 = (blk[:, 1:2] == eids).astype(jnp.int32)
        return tot + jnp.sum(oh0 + oh1, axis=0, keepdims=True)

    tot_row = jax.lax.fori_loop(0, NCH, cnt_body,
                                jnp.zeros((1, E), jnp.int32))
    tot_col = jnp.reshape(tot_row, (E, 1))

    er = jax.lax.broadcasted_iota(jnp.int32, (E, E), 0)
    ec = jax.lax.broadcasted_iota(jnp.int32, (E, E), 1)
    mask_lt = (er < ec).astype(jnp.float32)
    off_lo_row = jax.lax.dot_general(
        tot_col.astype(jnp.float32), mask_lt, (((0,), (0,)), ((), ())),
        preferred_element_type=jnp.float32,
        precision=jax.lax.Precision.HIGHEST)
    off_lo_col = jnp.reshape(off_lo_row, (E, 1)).astype(jnp.int32)
    off_hi_col = off_lo_col + tot_col

    r_i = jax.lax.broadcasted_iota(jnp.int32, (CH, CH), 0)
    c_i = jax.lax.broadcasted_iota(jnp.int32, (CH, CH), 1)
    tril_s = (c_i < r_i).astype(jnp.float32)
    base_row = off_lo_row

    def pb(c, carry):
        blk = sei_ref[pl.ds(c * CH, CH), :]
        oh0i = (blk[:, 0:1] == eids).astype(jnp.int32)
        oh1i = (blk[:, 1:2] == eids).astype(jnp.int32)
        oh0 = oh0i.astype(jnp.float32)
        oh1 = oh1i.astype(jnp.float32)
        A = jax.lax.dot_general(tril_s, oh0, (((1,), (0,)), ((), ())),
                                preferred_element_type=jnp.float32,
                                precision=jax.lax.Precision.HIGHEST)
        B = jax.lax.dot_general(tril_s, oh1, (((1,), (0,)), ((), ())),
                                preferred_element_type=jnp.float32,
                                precision=jax.lax.Precision.HIGHEST)
        base = base_row + carry.astype(jnp.float32)
        p0 = jnp.sum(oh0 * (base + A + B), axis=1, keepdims=True)
        p1 = jnp.sum(oh1 * (base + A + oh0 + B), axis=1, keepdims=True)
        inv0_ref[pl.ds(c * CH, CH), :] = p0.astype(jnp.int32)
        inv1_ref[pl.ds(c * CH, CH), :] = p1.astype(jnp.int32)
        return carry + jnp.sum(oh0i + oh1i, axis=0, keepdims=True)

    jax.lax.fori_loop(0, NCH, pb, jnp.zeros((1, E), jnp.int32))
    sc0_ref[...] = ts_ref[:, 0:1]
    sc1_ref[...] = ts_ref[:, 1:2]

    first_col = off_lo_col // T
    last_col = (off_hi_col - 1) // T
    tiles_col = jnp.where(tot_col > 0, last_col - first_col + 1, 0)
    mask_le_col = (ec <= er).astype(jnp.float32)
    cum_col = jax.lax.dot_general(
        mask_le_col, tiles_col.astype(jnp.float32),
        (((1,), (0,)), ((), ())),
        preferred_element_type=jnp.float32,
        precision=jax.lax.Precision.HIGHEST).astype(jnp.int32)
    total_b = cum_col[E - 1:E, :]

    grow = jax.lax.broadcasted_iota(jnp.int32, (1, 128), 1)
    ge_mask = (cum_col <= grow).astype(jnp.int32)
    e_of_g = jnp.sum(ge_mask, axis=0, keepdims=True)
    e_cl = jnp.minimum(e_of_g, E - 1)
    ecol = jax.lax.broadcasted_iota(jnp.int32, (E, 128), 0)
    ohg = (ecol == e_cl).astype(jnp.int32)

    def lk(v_col):
        return jnp.sum(ohg * v_col, axis=0, keepdims=True)

    first_g = lk(first_col)
    tiles_g = lk(tiles_col)
    cum_g = lk(cum_col)
    lo_g = lk(off_lo_col)
    hi_g = lk(off_hi_col)
    local = grow - (cum_g - tiles_g)
    t_g = first_g + local
    valid = grow < total_b
    tt = jnp.where(valid, t_g, NT - 1)
    eee = jnp.where(valid, e_cl, E - 1)
    st = jnp.where(valid, jnp.clip(lo_g - tt * T, 0, T), 0)
    en = jnp.where(valid, jnp.clip(hi_g - tt * T, 0, T), 0)
    meta_ref[0:1, :] = tt
    meta_ref[1:2, :] = eee
    meta_ref[2:3, :] = st
    meta_ref[3:4, :] = en
    meta_ref[4:5, :] = jnp.zeros((1, 128), jnp.int32)
    meta_ref[5:6, :] = jnp.zeros((1, 128), jnp.int32)
    meta_ref[6:7, :] = jnp.zeros((1, 128), jnp.int32)
    meta_ref[7:8, :] = jnp.zeros((1, 128), jnp.int32)


def _routing_tc(sei, ts):
    return pl.pallas_call(
        _routing_body,
        grid=(1,),
        in_specs=[
            pl.BlockSpec((N, K), lambda g: (0, 0)),
            pl.BlockSpec((N, K), lambda g: (0, 0)),
        ],
        out_specs=[
            pl.BlockSpec((N, 1), lambda g: (0, 0)),
            pl.BlockSpec((N, 1), lambda g: (0, 0)),
            pl.BlockSpec((N, 1), lambda g: (0, 0)),
            pl.BlockSpec((N, 1), lambda g: (0, 0)),
            pl.BlockSpec((8, 128), lambda g: (0, 0)),
        ],
        out_shape=[
            jax.ShapeDtypeStruct((N, 1), jnp.int32),
            jax.ShapeDtypeStruct((N, 1), jnp.int32),
            jax.ShapeDtypeStruct((N, 1), jnp.float32),
            jax.ShapeDtypeStruct((N, 1), jnp.float32),
            jax.ShapeDtypeStruct((8, 128), jnp.int32),
        ],
    )(sei, ts)


def _meta_jax(flat_exp):
    sizes = jnp.bincount(flat_exp, length=E).astype(jnp.int32)
    off = jnp.concatenate([jnp.zeros((1,), jnp.int32),
                           jnp.cumsum(sizes).astype(jnp.int32)])
    first_tile = off[:E] // T
    last_tile = (off[1:] - 1) // T
    tiles_e = jnp.where(sizes > 0, last_tile - first_tile + 1, 0).astype(jnp.int32)
    cum = jnp.cumsum(tiles_e)
    total = cum[-1]
    gids = jnp.arange(G, dtype=jnp.int32)
    e_of_g = jnp.searchsorted(cum, gids, side="right").astype(jnp.int32)
    valid = gids < total
    e_cl = jnp.minimum(e_of_g, E - 1)
    local = gids - (cum[e_cl] - tiles_e[e_cl])
    t_of_g = jnp.where(valid, first_tile[e_cl] + local, NT - 1).astype(jnp.int32)
    ee = jnp.where(valid, e_cl, E - 1).astype(jnp.int32)
    st = jnp.where(valid, jnp.clip(off[e_cl] - t_of_g * T, 0, T), 0).astype(jnp.int32)
    en = jnp.where(valid, jnp.clip(off[e_cl + 1] - t_of_g * T, 0, T), 0).astype(jnp.int32)
    meta = jnp.zeros((8, 128), jnp.int32)
    meta = meta.at[0, :G].set(t_of_g).at[1, :G].set(ee)
    meta = meta.at[2, :G].set(st).at[3, :G].set(en)
    return meta


def kernel(x, top_scores, selected_experts_indices, w1, w2, w3):
    inv0, inv1, sc0, sc1, meta = _routing_tc(selected_experts_indices,
                                             top_scores)
    inv0 = inv0.reshape(N)
    inv1 = inv1.reshape(N)

    ss = (jnp.zeros((NK,), jnp.float32)
          .at[inv0].set(sc0.reshape(N))
          .at[inv1].set(sc1.reshape(N)).reshape(NK, 1))

    rx = _build_dispatch()(x, inv0, inv1)

    ro = _grouped_swiglu(rx, ss, w1, w3, w2, meta)

    out = _build_combine()(ro, inv0, inv1)
    return out
